# baseline (speedup 1.0000x reference)
import functools

import jax
import jax.numpy as jnp
from jax import lax
from jax.experimental import pallas as pl
from jax.experimental.pallas import tpu as pltpu
from jax.experimental.pallas import tpu_sc as plsc

D = 64
NC = 2
NS = 16
NW = NC * NS
GROUP = 128
N_TOTAL = 4096 * 200
PER_W = N_TOTAL // NW
G = PER_W // GROUP


def _emb_body(x_hbm, table_hbm, out_hbm, idx_v, rows_v, sem_g):
    wid = lax.axis_index("s") * NC + lax.axis_index("c")
    pltpu.sync_copy(x_hbm.at[wid], idx_v)

    def grp(g, carry):
        pltpu.async_copy(table_hbm.at[idx_v.at[g]], rows_v, sem_g).wait()
        pltpu.sync_copy(rows_v, out_hbm.at[wid, g])
        return carry

    lax.fori_loop(0, G, grp, 0)


def kernel(x, table):
    x3 = x.reshape(NW, G, GROUP).astype(jnp.int32)
    mesh = plsc.VectorSubcoreMesh(core_axis_name="c", subcore_axis_name="s")
    out = pl.kernel(
        _emb_body,
        out_type=jax.ShapeDtypeStruct((NW, G, GROUP, D), jnp.float32),
        mesh=mesh,
        scratch_types=[
            pltpu.VMEM((G, GROUP), jnp.int32),
            pltpu.VMEM((GROUP, D), jnp.float32),
            pltpu.SemaphoreType.DMA,
        ],
        compiler_params=pltpu.CompilerParams(use_tc_tiling_on_sc=False),
    )(x3, table)
    return out.reshape(4096, 200, D)


# speedup vs baseline: 1.1168x; 1.1168x over previous
import jax
import jax.numpy as jnp
from jax import lax
from jax.experimental import pallas as pl
from jax.experimental.pallas import tpu as pltpu
from jax.experimental.pallas import tpu_sc as plsc

D = 64
NC = 2
NS = 16
NW = NC * NS
GROUP = 128
N_TOTAL = 4096 * 200
PER_W = N_TOTAL // NW
G = PER_W // GROUP
K = 4
CHUNK = K * GROUP
C = G // K


def _emb_body(x_hbm, table_hbm, out_hbm, idx_v, buf, sem_g, sem_s):
    wid = lax.axis_index("s") * NC + lax.axis_index("c")
    pltpu.sync_copy(x_hbm.at[wid], idx_v)

    def fire_chunk(c, p):
        for j in range(K):
            pltpu.async_copy(
                table_hbm.at[idx_v.at[c * K + j]],
                buf.at[p, pl.ds(j * GROUP, GROUP)],
                sem_g,
            )

    def wait_chunk(p):
        pltpu.make_async_copy(
            table_hbm.at[pl.ds(0, CHUNK)], buf.at[p], sem_g
        ).wait()

    fire_chunk(0, 0)

    def chunk(c, carry):
        p = lax.rem(c, 2)
        @pl.when(c >= 1)
        def _():
            pltpu.make_async_copy(buf.at[1 - p], out_hbm.at[wid, 0], sem_s).wait()

        @pl.when(c + 1 < C)
        def _():
            fire_chunk(c + 1, 1 - p)

        wait_chunk(p)
        pltpu.async_copy(buf.at[p], out_hbm.at[wid, c], sem_s)
        return carry

    lax.fori_loop(0, C, chunk, 0)
    pltpu.make_async_copy(buf.at[0], out_hbm.at[wid, 0], sem_s).wait()


def kernel(x, table):
    x3 = x.reshape(NW, G, GROUP).astype(jnp.int32)
    mesh = plsc.VectorSubcoreMesh(core_axis_name="c", subcore_axis_name="s")
    out = pl.kernel(
        _emb_body,
        out_type=jax.ShapeDtypeStruct((NW, C, CHUNK, D), jnp.float32),
        mesh=mesh,
        scratch_types=[
            pltpu.VMEM((G, GROUP), jnp.int32),
            pltpu.VMEM((2, CHUNK, D), jnp.float32),
            pltpu.SemaphoreType.DMA,
            pltpu.SemaphoreType.DMA,
        ],
        compiler_params=pltpu.CompilerParams(use_tc_tiling_on_sc=False),
    )(x3, table)
    return out.reshape(4096, 200, D)
